# Initial kernel scaffold; baseline (speedup 1.0000x reference)
import jax
import jax.numpy as jnp
from jax.experimental import pallas as pl


def kernel(x, edge_index, batch, target, W1, a_s1, a_d1, b1, W2, a_s2, a_d2, b2, fg_W, fg_b, emb, cW, cb, ftW, ftb, f1W, f1b, f2W, f2b, oW, ob):
    raise NotImplementedError("write your pallas kernel here")


# speedup vs baseline: 1.0352x; 1.0352x over previous
import functools

import jax
import jax.numpy as jnp
from jax import lax
from jax.experimental import pallas as pl

F32 = jnp.float32


def _matmul_body(x_ref, w_ref, o_ref):
    o_ref[...] = jnp.dot(x_ref[...], w_ref[...],
                         preferred_element_type=F32, precision=lax.Precision.HIGHEST)


def _matmul(x, w, bm):
    m, k = x.shape
    n = w.shape[1]
    assert m % bm == 0, (m, bm)
    return pl.pallas_call(
        _matmul_body,
        grid=(m // bm,),
        in_specs=[
            pl.BlockSpec((bm, k), lambda i: (i, 0)),
            pl.BlockSpec((k, n), lambda i: (0, 0)),
        ],
        out_specs=pl.BlockSpec((bm, n), lambda i: (i, 0)),
        out_shape=jax.ShapeDtypeStruct((m, n), F32),
    )(x, w)


def _conv_body(tgt_ref, emb_ref, cwt_ref, cb_ref, o_ref, *, L, KSZ, GB):
    embp = emb_ref[...]
    vp = embp.shape[0]
    cb = cb_ref[...]
    for g in range(GB):
        tgt = tgt_ref[g, 0, :]
        oh = (tgt[:, None] == lax.broadcasted_iota(jnp.int32, (L, vp), 1))
        z = jnp.dot(oh.astype(F32), embp, preferred_element_type=F32, precision=lax.Precision.HIGHEST)
        lo = L - KSZ + 1
        acc = jnp.zeros((lo, cwt_ref.shape[2]), F32)
        for k in range(KSZ):
            acc += jnp.dot(z[k:k + lo, :], cwt_ref[k],
                           preferred_element_type=F32, precision=lax.Precision.HIGHEST)
        o_ref[g, :] = jnp.maximum(jnp.max(acc, axis=0) + cb[0], 0.0)


def _protein_conv(target, emb, cW, cb, gb):
    B, L = target.shape
    vocab, D = emb.shape
    vp = (vocab + 7) // 8 * 8
    embp = jnp.pad(emb, ((0, vp - vocab), (0, 0)))
    cwt = jnp.transpose(cW, (2, 1, 0))
    ksz = cW.shape[2]
    co = cW.shape[0]
    tgt3 = target.reshape(B, 1, L)
    body = functools.partial(_conv_body, L=L, KSZ=ksz, GB=gb)
    return pl.pallas_call(
        body,
        grid=(B // gb,),
        in_specs=[
            pl.BlockSpec((gb, 1, L), lambda i: (i, 0, 0)),
            pl.BlockSpec((vp, D), lambda i: (0, 0)),
            pl.BlockSpec((ksz, D, co), lambda i: (0, 0, 0)),
            pl.BlockSpec((1, co), lambda i: (0, 0)),
        ],
        out_specs=pl.BlockSpec((gb, co), lambda i: (i, 0)),
        out_shape=jax.ShapeDtypeStruct((B, co), F32),
    )(tgt3, embp, cwt, cb.reshape(1, co))


def _head_body(graw_ref, z_ref, fgw_ref, fgb_ref, ftw_ref, ftb_ref,
               f1wg_ref, f1wx_ref, f1b_ref, f2w_ref, f2b_ref,
               ow_ref, ob_ref, o_ref):
    g = jnp.maximum(jnp.dot(graw_ref[...], fgw_ref[...],
                            preferred_element_type=F32, precision=lax.Precision.HIGHEST) + fgb_ref[...], 0.0)
    xt = jnp.dot(z_ref[...], ftw_ref[...],
                 preferred_element_type=F32, precision=lax.Precision.HIGHEST) + ftb_ref[...]
    t1 = jnp.dot(g, f1wg_ref[...], preferred_element_type=F32, precision=lax.Precision.HIGHEST)
    t1 += jnp.dot(xt, f1wx_ref[...], preferred_element_type=F32, precision=lax.Precision.HIGHEST)
    t1 = jnp.maximum(t1 + f1b_ref[...], 0.0)
    t2 = jnp.maximum(jnp.dot(t1, f2w_ref[...],
                             preferred_element_type=F32, precision=lax.Precision.HIGHEST) + f2b_ref[...], 0.0)
    o_ref[...] = jnp.dot(t2, ow_ref[...],
                         preferred_element_type=F32, precision=lax.Precision.HIGHEST) + ob_ref[...]


def _head(graw, z, fg_W, fg_b, ftW, ftb, f1W, f1b, f2W, f2b, oW, ob):
    B, D = graw.shape
    f1Wg = f1W[:D]
    f1Wx = f1W[D:]
    full = lambda *s: pl.BlockSpec(s, lambda: tuple(0 for _ in s))
    args = (graw, z, fg_W, fg_b.reshape(1, -1), ftW, ftb.reshape(1, -1),
            f1Wg, f1Wx, f1b.reshape(1, -1), f2W, f2b.reshape(1, -1),
            oW, ob.reshape(1, 1))
    return pl.pallas_call(
        _head_body,
        in_specs=[full(*a.shape) for a in args],
        out_specs=full(B, 1),
        out_shape=jax.ShapeDtypeStruct((B, 1), F32),
    )(*args)


def _edge_softmax_agg(feat, es, ed, src, dst, n):
    e = jax.nn.leaky_relu(es[src] + ed[dst], 0.2)
    emax = jax.ops.segment_max(e, dst, num_segments=n)
    emax = jnp.where(jnp.isfinite(emax), emax, 0.0)
    ee = jnp.exp(e - emax[dst])
    den = jax.ops.segment_sum(ee, dst, num_segments=n)
    alpha = ee / (den[dst] + 1e-16)
    msg = feat[src][:, None, :] * alpha[..., None]
    return jax.ops.segment_sum(msg, dst, num_segments=n)


def kernel(x, edge_index, batch, target, W1, a_s1, a_d1, b1, W2, a_s2, a_d2,
           b2, fg_W, fg_b, emb, cW, cb, ftW, ftb, f1W, f1b, f2W, f2b, oW, ob):
    n, fin = x.shape
    heads = a_s1.shape[0]
    loop = jnp.arange(n, dtype=edge_index.dtype)
    src = jnp.concatenate([edge_index[0], loop])
    dst = jnp.concatenate([edge_index[1], loop])

    W1r = W1.reshape(fin, heads, fin)
    W1s = jnp.einsum('ihc,hc->ih', W1r, a_s1, precision=lax.Precision.HIGHEST)
    W1d = jnp.einsum('ihc,hc->ih', W1r, a_d1, precision=lax.Precision.HIGHEST)
    hp = (heads + 7) // 8 * 8
    zpad = jnp.zeros((fin, hp - heads), F32)
    cat1 = jnp.concatenate([W1s, zpad, W1d, zpad], 1)
    esd1 = _matmul(x, cat1, 1000)
    es1, ed1 = esd1[:, :heads], esd1[:, hp:hp + heads]

    agg1 = _edge_softmax_agg(x, es1, ed1, src, dst, n)
    out1 = jnp.einsum('nhf,fhc->nhc', agg1, W1r, precision=lax.Precision.HIGHEST).reshape(n, heads * fin)
    h1 = jax.nn.elu(out1 + b1)

    od = W2.shape[1]
    cat2 = jnp.concatenate([W2, jnp.dot(W2, a_s2.reshape(od, 1), precision=lax.Precision.HIGHEST),
                            jnp.dot(W2, a_d2.reshape(od, 1), precision=lax.Precision.HIGHEST),
                            jnp.zeros((W2.shape[0], 6), F32)], 1)
    h2e = _matmul(h1, cat2, 1000)
    h2 = h2e[:, :od]
    es2 = h2e[:, od:od + 1]
    ed2 = h2e[:, od + 1:od + 2]

    agg2 = _edge_softmax_agg(h2, es2, ed2, src, dst, n)
    h2o = jnp.maximum(agg2[:, 0, :] + b2, 0.0)

    nb = target.shape[0]
    graw = jax.ops.segment_max(h2o, batch, num_segments=nb)
    graw = jnp.where(jnp.isfinite(graw), graw, 0.0)

    z = _protein_conv(target, emb, cW, cb, 8)

    return _head(graw, z, fg_W, fg_b, ftW, ftb, f1W, f1b, f2W, f2b, oW, ob)
